# Initial kernel scaffold; baseline (speedup 1.0000x reference)
import jax
import jax.numpy as jnp
from jax.experimental import pallas as pl


def kernel(inputs, W1, b1, W2, b2):
    raise NotImplementedError("write your pallas kernel here")


# speedup vs baseline: 9.4757x; 9.4757x over previous
import jax
import jax.numpy as jnp
from jax import lax
from jax.experimental import pallas as pl
from jax.experimental.pallas import tpu as pltpu

_K_TOP = 256
_CELLS_PER_ROW = 8


def _cellcnn_body(xw_ref, bd_ref, b1_ref, w2_ref, b2_ref, out_ref, h_ref):
    nr, nl = h_ref.shape
    nf = nl // _CELLS_PER_ROW

    x = xw_ref[0]
    h = jnp.dot(x, bd_ref[...], preferred_element_type=jnp.float32)
    h = jnp.maximum(h + b1_ref[...], 0.0)
    h_ref[...] = h

    li = lax.broadcasted_iota(jnp.int32, (nl, nl), 0)
    mi = lax.broadcasted_iota(jnp.int32, (nl, nl), 1)
    foldm = jnp.where((li % nf) == (mi % nf), 1.0, 0.0).astype(jnp.float32)

    def count_ge(t_bits):
        t = lax.bitcast_convert_type(t_bits, jnp.float32)
        mask = (h_ref[...] >= t).astype(jnp.float32)
        cnt = jnp.sum(mask, axis=0, keepdims=True)
        return jnp.dot(cnt, foldm, preferred_element_type=jnp.float32)

    def bs_body(_, carry):
        lo, hi = carry
        mid = lo + lax.div(hi - lo, 2)
        pred = count_ge(mid) >= float(_K_TOP)
        return jnp.where(pred, mid, lo), jnp.where(pred, hi, mid)

    lo0 = jnp.zeros((1, nl), jnp.int32)
    hi0 = jnp.full((1, nl), jnp.int32(2**31 - 1))
    lo, hi = lax.fori_loop(0, 31, bs_body, (lo0, hi0))

    t_lo = lax.bitcast_convert_type(lo, jnp.float32)
    t_hi = lax.bitcast_convert_type(hi, jnp.float32)
    hh = h_ref[...]
    mgt = hh >= t_hi
    sums = jnp.sum(jnp.where(mgt, hh, 0.0), axis=0, keepdims=True)
    cgt = jnp.sum(mgt.astype(jnp.float32), axis=0, keepdims=True)
    sumsf = jnp.dot(sums, foldm, preferred_element_type=jnp.float32)
    cgtf = jnp.dot(cgt, foldm, preferred_element_type=jnp.float32)
    sum_top = sumsf + (float(_K_TOP) - cgtf) * t_lo
    pooled = sum_top[:, :nf] * (1.0 / _K_TOP)

    z = jnp.sum(pooled * w2_ref[...], axis=1, keepdims=True) + b2_ref[...]
    out_ref[0] = 1.0 / (1.0 + jnp.exp(-z))


def _build_call(B, NR, D, F):
    C = _CELLS_PER_ROW
    return pl.pallas_call(
        _cellcnn_body,
        grid=(B,),
        in_specs=[
            pl.BlockSpec((1, NR, C * D), lambda b: (b, 0, 0)),
            pl.BlockSpec((C * D, C * F), lambda b: (0, 0)),
            pl.BlockSpec((1, C * F), lambda b: (0, 0)),
            pl.BlockSpec((1, F), lambda b: (0, 0)),
            pl.BlockSpec((1, 1), lambda b: (0, 0)),
        ],
        out_specs=pl.BlockSpec((1, 1, 1), lambda b: (b, 0, 0)),
        out_shape=jax.ShapeDtypeStruct((B, 1, 1), jnp.float32),
        scratch_shapes=[pltpu.VMEM((NR, C * F), jnp.float32)],
    )


def kernel(inputs, W1, b1, W2, b2):
    B, N, D = inputs.shape
    F = W1.shape[1]
    C = _CELLS_PER_ROW
    NR = N // C
    xw = inputs.reshape(B, NR, C * D)
    eye = jnp.eye(C, dtype=W1.dtype)
    bd = jnp.einsum("ce,df->cdef", eye, W1).reshape(C * D, C * F)
    b1t = jnp.tile(b1, C).reshape(1, C * F)
    w2t = W2.reshape(1, F)
    b2r = b2.reshape(1, 1)
    out = _build_call(B, NR, D, F)(xw, bd, b1t, w2t, b2r)
    return out.reshape(B, 1)


# speedup vs baseline: 14.8009x; 1.5620x over previous
import jax
import jax.numpy as jnp
from jax import lax
from jax.experimental import pallas as pl
from jax.experimental.pallas import tpu as pltpu

_K_TOP = 256
_CELLS_PER_ROW = 8


def _cellcnn_body(xw_ref, bd_ref, b1_ref, w2_ref, b2_ref, out_ref, hall_ref):
    B, nr, nl = hall_ref.shape
    nf = nl // _CELLS_PER_ROW
    step = pl.program_id(0)

    @pl.when(step < B)
    def _matmul():
        x = xw_ref[0]
        h = jnp.dot(x, bd_ref[...], preferred_element_type=jnp.float32)
        hall_ref[step] = jnp.maximum(h + b1_ref[...], 0.0)

    @pl.when(step == B)
    def _search():
        li = lax.broadcasted_iota(jnp.int32, (nl, nl), 0)
        mi = lax.broadcasted_iota(jnp.int32, (nl, nl), 1)
        foldm = jnp.where((li % nf) == (mi % nf), 1.0, 0.0).astype(jnp.float32)

        def count_ge(t_bits):
            t = lax.bitcast_convert_type(t_bits, jnp.float32)
            cnt = jnp.concatenate(
                [
                    jnp.sum(
                        (hall_ref[b] >= t[b : b + 1]).astype(jnp.float32),
                        axis=0,
                        keepdims=True,
                    )
                    for b in range(B)
                ],
                axis=0,
            )
            return jnp.dot(cnt, foldm, preferred_element_type=jnp.float32)

        def bs_body(_, carry):
            lo, hi = carry
            mid = lo + lax.div(hi - lo, 2)
            pred = count_ge(mid) >= float(_K_TOP)
            return jnp.where(pred, mid, lo), jnp.where(pred, hi, mid)

        lo0 = jnp.zeros((B, nl), jnp.int32)
        hi0 = jnp.full((B, nl), jnp.int32(2**31 - 1))
        lo, hi = lax.fori_loop(0, 31, bs_body, (lo0, hi0))

        t_lo = lax.bitcast_convert_type(lo, jnp.float32)
        t_hi = lax.bitcast_convert_type(hi, jnp.float32)
        sums_l, cgt_l = [], []
        for b in range(B):
            hh = hall_ref[b]
            mgt = hh >= t_hi[b : b + 1]
            sums_l.append(
                jnp.sum(jnp.where(mgt, hh, 0.0), axis=0, keepdims=True)
            )
            cgt_l.append(
                jnp.sum(mgt.astype(jnp.float32), axis=0, keepdims=True)
            )
        sums = jnp.concatenate(sums_l, axis=0)
        cgt = jnp.concatenate(cgt_l, axis=0)
        sumsf = jnp.dot(sums, foldm, preferred_element_type=jnp.float32)
        cgtf = jnp.dot(cgt, foldm, preferred_element_type=jnp.float32)
        sum_top = sumsf + (float(_K_TOP) - cgtf) * t_lo
        pooled = sum_top[:, :nf] * (1.0 / _K_TOP)

        z = jnp.sum(pooled * w2_ref[...], axis=1, keepdims=True) + b2_ref[...]
        out_ref[...] = (1.0 / (1.0 + jnp.exp(-z))).reshape(B, 1, 1)


def _build_call(B, NR, D, F):
    C = _CELLS_PER_ROW
    return pl.pallas_call(
        _cellcnn_body,
        grid=(B + 1,),
        in_specs=[
            pl.BlockSpec((1, NR, C * D), lambda b: (jnp.minimum(b, B - 1), 0, 0)),
            pl.BlockSpec((C * D, C * F), lambda b: (0, 0)),
            pl.BlockSpec((1, C * F), lambda b: (0, 0)),
            pl.BlockSpec((1, F), lambda b: (0, 0)),
            pl.BlockSpec((1, 1), lambda b: (0, 0)),
        ],
        out_specs=pl.BlockSpec((B, 1, 1), lambda b: (0, 0, 0)),
        out_shape=jax.ShapeDtypeStruct((B, 1, 1), jnp.float32),
        scratch_shapes=[pltpu.VMEM((B, NR, C * F), jnp.float32)],
    )


def kernel(inputs, W1, b1, W2, b2):
    B, N, D = inputs.shape
    F = W1.shape[1]
    C = _CELLS_PER_ROW
    NR = N // C
    xw = inputs.reshape(B, NR, C * D)
    eye = jnp.eye(C, dtype=W1.dtype)
    bd = jnp.einsum("ce,df->cdef", eye, W1).reshape(C * D, C * F)
    b1t = jnp.tile(b1, C).reshape(1, C * F)
    w2t = W2.reshape(1, F)
    b2r = b2.reshape(1, 1)
    out = _build_call(B, NR, D, F)(xw, bd, b1t, w2t, b2r)
    return out.reshape(B, 1)
